# baseline (speedup 1.0000x reference)
import functools

import jax
import jax.numpy as jnp
from jax import lax
from jax.experimental import pallas as pl
from jax.experimental.pallas import tpu as pltpu
from jax.experimental.pallas import tpu_sc as plsc

_INFO = plsc.get_sparse_core_info()
_NC = _INFO.num_cores
_NS = _INFO.num_subcores
_NW = _NC * _NS
_CH = 128


@functools.partial(jax.jit, static_argnums=())
def _sc_gather(table, idx3):
    nw, n_chunks, ch = idx3.shape
    b_per_w = n_chunks * ch
    b = nw * b_per_w
    d = table.shape[1]
    mesh = plsc.VectorSubcoreMesh(core_axis_name="c", subcore_axis_name="s")

    @functools.partial(
        pl.kernel,
        mesh=mesh,
        out_type=jax.ShapeDtypeStruct((b, d), jnp.float32),
        scratch_types=[
            pltpu.VMEM((n_chunks, ch), jnp.int32),
            pltpu.VMEM((b_per_w, d), jnp.float32),
            pltpu.SemaphoreType.DMA,
        ],
        compiler_params=pltpu.CompilerParams(use_tc_tiling_on_sc=False),
    )
    def k(table_hbm, idx_hbm, out_hbm, idx_v, rows_v, sem):
        wid = lax.axis_index("s") * _NC + lax.axis_index("c")
        base = wid * b_per_w
        pltpu.sync_copy(idx_hbm.at[wid], idx_v)
        copies = []
        for j in range(n_chunks):
            copies.append(
                pltpu.async_copy(
                    table_hbm.at[idx_v.at[j]],
                    rows_v.at[pl.ds(j * ch, ch)],
                    sem,
                )
            )
        for cp in copies:
            cp.wait()
        pltpu.sync_copy(rows_v, out_hbm.at[pl.ds(base, b_per_w)])

    return k(table, idx3)


def kernel(g, h, r, norm, node_attri):
    idx = h.reshape(-1).astype(jnp.int32)
    b = idx.shape[0]
    n_chunks = b // (_NW * _CH)
    idx3 = idx.reshape(_NW, n_chunks, _CH)
    return _sc_gather(node_attri, idx3)


# speedup vs baseline: 6.0471x; 6.0471x over previous
import functools

import jax
import jax.numpy as jnp
from jax import lax
from jax.experimental import pallas as pl
from jax.experimental.pallas import tpu as pltpu
from jax.experimental.pallas import tpu_sc as plsc

_INFO = plsc.get_sparse_core_info()
_NC = _INFO.num_cores
_NS = _INFO.num_subcores
_NW = _NC * _NS
_GRP = 16


def _sc_gather_blocks(tab3, idx2):
    nw, b_per_w = idx2.shape
    b = nw * b_per_w
    mesh = plsc.VectorSubcoreMesh(core_axis_name="c", subcore_axis_name="s")
    n_out_blk = b_per_w // 128
    n_grp = b_per_w // _GRP

    @functools.partial(
        pl.kernel,
        mesh=mesh,
        out_type=jax.ShapeDtypeStruct((2, 8, b), jnp.float32),
        scratch_types=[
            pltpu.VMEM((b_per_w,), jnp.int32),
            pltpu.VMEM((2 * _GRP, 2, 8, 128), jnp.float32),
            pltpu.VMEM((2, n_out_blk, 8, 128), jnp.float32),
            pltpu.SemaphoreType.DMA,
            pltpu.SemaphoreType.DMA,
        ],
        compiler_params=pltpu.CompilerParams(
            disable_bounds_checks=True, needs_layout_passes=False
        ),
    )
    def k(tab_hbm, idx_hbm, out_hbm, idx_v, blks, rows4, sem_a, sem_b):
        wid = lax.axis_index("s") * _NC + lax.axis_index("c")
        base = wid * b_per_w
        pltpu.sync_copy(idx_hbm.at[wid], idx_v)
        half_sems = (sem_a, sem_b)

        def issue_group(g_scalar, half):
            v16 = idx_v[pl.ds(g_scalar * _GRP, _GRP)]
            for bi in range(_GRP):
                c0 = pl.multiple_of((v16[bi] // 128) * 128, 128)
                pltpu.async_copy(
                    tab_hbm.at[:, :, pl.ds(c0, 128)],
                    blks.at[half * _GRP + bi],
                    half_sems[half],
                )

        def process_group(g_scalar, half):
            for bi in range(_GRP):
                pltpu.make_async_copy(
                    tab_hbm.at[:, :, pl.ds(0, 128)],
                    blks.at[half * _GRP + bi],
                    half_sems[half],
                ).wait()
            v16 = idx_v[pl.ds(g_scalar * _GRP, _GRP)]
            l_vec = v16 % 128
            slot = lax.iota(jnp.int32, 16) + half * _GRP
            j0 = g_scalar * _GRP
            tcc = j0 // 128
            l2 = j0 % 128
            for d in range(16):
                tr_b = jnp.full((16,), d // 8, jnp.int32)
                s_b = jnp.full((16,), d % 8, jnp.int32)
                vals = plsc.load_gather(blks, [slot, tr_b, s_b, l_vec])
                rows4[d // 8, tcc, d % 8, pl.ds(l2, _GRP)] = vals

        issue_group(0, 0)
        issue_group(1, 1)

        def body(g2, carry):
            g = g2 * 2
            process_group(g, 0)

            @pl.when(g + 2 < n_grp)
            def _():
                issue_group(g + 2, 0)

            process_group(g + 1, 1)

            @pl.when(g + 3 < n_grp)
            def _():
                issue_group(g + 3, 1)

            return carry

        lax.fori_loop(0, n_grp // 2, body, 0)

        for tr in range(2):
            for tcc in range(n_out_blk):
                pltpu.sync_copy(
                    rows4.at[tr, tcc],
                    out_hbm.at[tr].at[:, pl.ds(base + tcc * 128, 128)],
                )

    return k(tab3, idx2)


def kernel(g, h, r, norm, node_attri):
    idx = h.reshape(-1).astype(jnp.int32)
    b = idx.shape[0]
    idx2 = idx.reshape(_NW, b // _NW)
    tab3 = node_attri.T.reshape(2, 8, node_attri.shape[0])
    out3 = _sc_gather_blocks(tab3, idx2)
    return out3.reshape(16, b).T


# speedup vs baseline: 6.2918x; 1.0405x over previous
import functools

import jax
import jax.numpy as jnp
from jax import lax
from jax.experimental import pallas as pl
from jax.experimental.pallas import tpu as pltpu
from jax.experimental.pallas import tpu_sc as plsc

_INFO = plsc.get_sparse_core_info()
_NC = _INFO.num_cores
_NS = _INFO.num_subcores
_NW = _NC * _NS
_GRP = 16


_NPH = 3


def _sc_gather_blocks(tab3, idx1):
    (b,) = idx1.shape
    b_per_w = b // _NW
    mesh = plsc.VectorSubcoreMesh(core_axis_name="c", subcore_axis_name="s")
    n_out_blk = b_per_w // 128
    n_grp = b_per_w // _GRP
    n_tail = n_grp % _NPH
    n_main = n_grp - n_tail

    @functools.partial(
        pl.kernel,
        mesh=mesh,
        out_type=jax.ShapeDtypeStruct((2, 8, b), jnp.float32),
        scratch_types=(
            [
                pltpu.VMEM((b_per_w,), jnp.int32),
                pltpu.VMEM((_NPH * _GRP, 2, 8, 128), jnp.float32),
                pltpu.VMEM((2, n_out_blk, 8, 128), jnp.float32),
            ]
            + [pltpu.SemaphoreType.DMA] * _NPH
        ),
        compiler_params=pltpu.CompilerParams(
            disable_bounds_checks=True, needs_layout_passes=False
        ),
    )
    def k(tab_hbm, idx_hbm, out_hbm, idx_v, blks, rows4, *sems):
        wid = lax.axis_index("s") * _NC + lax.axis_index("c")
        base = wid * b_per_w
        pltpu.sync_copy(idx_hbm.at[pl.ds(base, b_per_w)], idx_v)

        def issue_group(g_scalar, ph):
            v16 = idx_v[pl.ds(g_scalar * _GRP, _GRP)]
            for bi in range(_GRP):
                c0 = pl.multiple_of((v16[bi] // 128) * 128, 128)
                pltpu.async_copy(
                    tab_hbm.at[:, :, pl.ds(c0, 128)],
                    blks.at[ph * _GRP + bi],
                    sems[ph],
                )

        def process_group(g_scalar, ph):
            for bi in range(_GRP):
                pltpu.make_async_copy(
                    tab_hbm.at[:, :, pl.ds(0, 128)],
                    blks.at[ph * _GRP + bi],
                    sems[ph],
                ).wait()
            v16 = idx_v[pl.ds(g_scalar * _GRP, _GRP)]
            l_vec = v16 % 128
            slot = lax.iota(jnp.int32, 16) + ph * _GRP
            j0 = g_scalar * _GRP
            tcc = j0 // 128
            l2 = j0 % 128
            for d in range(16):
                tr_b = jnp.full((16,), d // 8, jnp.int32)
                s_b = jnp.full((16,), d % 8, jnp.int32)
                vals = plsc.load_gather(blks, [slot, tr_b, s_b, l_vec])
                rows4[d // 8, tcc, d % 8, pl.ds(l2, _GRP)] = vals

        for ph in range(_NPH):
            issue_group(ph, ph)

        def body(gk, carry):
            g = gk * _NPH
            for ph in range(_NPH):
                process_group(g + ph, ph)

                @pl.when(g + ph + _NPH < n_grp)
                def _():
                    issue_group(g + ph + _NPH, ph)

            return carry

        lax.fori_loop(0, n_main // _NPH, body, 0)
        for t in range(n_tail):
            process_group(n_main + t, t)

        for tr in range(2):
            for tcc in range(n_out_blk):
                pltpu.sync_copy(
                    rows4.at[tr, tcc],
                    out_hbm.at[tr].at[:, pl.ds(base + tcc * 128, 128)],
                )

    return k(tab3, idx1)


def kernel(g, h, r, norm, node_attri):
    idx = h.reshape(-1).astype(jnp.int32)
    tab3 = node_attri.T.reshape(2, 8, node_attri.shape[0])
    out3 = _sc_gather_blocks(tab3, idx)
    return out3.reshape(16, idx.shape[0]).T


# speedup vs baseline: 7.2660x; 1.1548x over previous
import functools

import jax
import jax.numpy as jnp
from jax import lax
from jax.experimental import pallas as pl
from jax.experimental.pallas import tpu as pltpu
from jax.experimental.pallas import tpu_sc as plsc

_INFO = plsc.get_sparse_core_info()
_NC = _INFO.num_cores
_NS = _INFO.num_subcores
_NW = _NC * _NS
_GRP = 16
_NPH = 3
_BW = 16


def _sc_gather_blocks(tab3, idx1):
    (b,) = idx1.shape
    b_per_w = b // _NW
    mesh = plsc.VectorSubcoreMesh(core_axis_name="c", subcore_axis_name="s")
    n_out_blk = b_per_w // 128
    n_grp = b_per_w // _GRP
    n_tail = n_grp % _NPH
    n_main = n_grp - n_tail

    @functools.partial(
        pl.kernel,
        mesh=mesh,
        out_type=jax.ShapeDtypeStruct((2, 8, b), jnp.float32),
        scratch_types=(
            [
                pltpu.VMEM((b_per_w,), jnp.int32),
                pltpu.VMEM((_NPH * 2, 2, 8, 128), jnp.float32),
                pltpu.VMEM((2, n_out_blk, 8, 128), jnp.float32),
            ]
            + [pltpu.SemaphoreType.DMA] * _NPH
        ),
        compiler_params=pltpu.CompilerParams(
            disable_bounds_checks=True, needs_layout_passes=False
        ),
    )
    def k(tab_hbm, idx_hbm, out_hbm, idx_v, blks, rows4, *sems):
        wid = lax.axis_index("s") * _NC + lax.axis_index("c")
        base = wid * b_per_w
        pltpu.sync_copy(idx_hbm.at[pl.ds(base, b_per_w)], idx_v)

        def issue_group(g_scalar, ph):
            v16 = idx_v[pl.ds(g_scalar * _GRP, _GRP)]
            for bi in range(_GRP):
                c0 = pl.multiple_of((v16[bi] // _BW) * _BW, _BW)
                pltpu.async_copy(
                    tab_hbm.at[:, :, pl.ds(c0, _BW)],
                    blks.at[ph * 2 + bi // 8].at[
                        :, :, pl.ds((bi % 8) * _BW, _BW)
                    ],
                    sems[ph],
                )

        def process_group(g_scalar, ph):
            for bi in range(_GRP):
                pltpu.make_async_copy(
                    tab_hbm.at[:, :, pl.ds(0, _BW)],
                    blks.at[ph * 2 + bi // 8].at[
                        :, :, pl.ds((bi % 8) * _BW, _BW)
                    ],
                    sems[ph],
                ).wait()
            v16 = idx_v[pl.ds(g_scalar * _GRP, _GRP)]
            k16 = lax.iota(jnp.int32, 16)
            l_vec = (k16 % 8) * _BW + v16 % _BW
            slot = ph * 2 + k16 // 8
            j0 = g_scalar * _GRP
            tcc = j0 // 128
            l2 = j0 % 128
            for d in range(16):
                tr_b = jnp.full((16,), d // 8, jnp.int32)
                s_b = jnp.full((16,), d % 8, jnp.int32)
                vals = plsc.load_gather(blks, [slot, tr_b, s_b, l_vec])
                rows4[d // 8, tcc, d % 8, pl.ds(l2, _GRP)] = vals

        for ph in range(_NPH):
            issue_group(ph, ph)

        def body(gk, carry):
            g = gk * _NPH
            for ph in range(_NPH):
                process_group(g + ph, ph)

                @pl.when(g + ph + _NPH < n_grp)
                def _():
                    issue_group(g + ph + _NPH, ph)

            return carry

        lax.fori_loop(0, n_main // _NPH, body, 0)
        for t in range(n_tail):
            process_group(n_main + t, t)

        for tr in range(2):
            for tcc in range(n_out_blk):
                pltpu.sync_copy(
                    rows4.at[tr, tcc],
                    out_hbm.at[tr].at[:, pl.ds(base + tcc * 128, 128)],
                )

    return k(tab3, idx1)


def kernel(g, h, r, norm, node_attri):
    idx = h.reshape(-1).astype(jnp.int32)
    tab3 = node_attri.T.reshape(2, 8, node_attri.shape[0])
    out3 = _sc_gather_blocks(tab3, idx)
    return out3.reshape(16, idx.shape[0]).T


# speedup vs baseline: 8.3173x; 1.1447x over previous
import functools

import jax
import jax.numpy as jnp
from jax import lax
from jax.experimental import pallas as pl
from jax.experimental.pallas import tpu as pltpu
from jax.experimental.pallas import tpu_sc as plsc

_INFO = plsc.get_sparse_core_info()
_NC = _INFO.num_cores
_NS = _INFO.num_subcores
_NW = _NC * _NS
_GRP = 16
_NPH = 3
_BW = 16


def _sc_gather_blocks(tab3, idx1):
    (b,) = idx1.shape
    b_per_w = b // _NW
    mesh = plsc.VectorSubcoreMesh(core_axis_name="c", subcore_axis_name="s")
    n_out_blk = b_per_w // 128
    n_grp = b_per_w // _GRP
    n_tail = n_grp % _NPH
    n_main = n_grp - n_tail

    @functools.partial(
        pl.kernel,
        mesh=mesh,
        out_type=jax.ShapeDtypeStruct((2, 8, b), jnp.float32),
        scratch_types=(
            [
                pltpu.VMEM((b_per_w,), jnp.int32),
                pltpu.VMEM((_NPH * 2, 2, 8, 128), jnp.float32),
                pltpu.VMEM((2, n_out_blk, 8, 128), jnp.float32),
            ]
            + [pltpu.SemaphoreType.DMA] * _NPH
        ),
        compiler_params=pltpu.CompilerParams(
            disable_bounds_checks=True, needs_layout_passes=False
        ),
    )
    def k(tab_hbm, idx_hbm, out_hbm, idx_v, blks, rows4, *sems):
        wid = lax.axis_index("s") * _NC + lax.axis_index("c")
        base = wid * b_per_w
        pltpu.sync_copy(idx_hbm.at[pl.ds(base, b_per_w)], idx_v)

        def issue_group(g_scalar, ph):
            c16 = (idx_v[pl.ds(g_scalar * _GRP, _GRP)] // _BW) * _BW
            for bi in range(_GRP):
                c0 = pl.multiple_of(c16[bi], _BW)
                pltpu.async_copy(
                    tab_hbm.at[:, :, pl.ds(c0, _BW)],
                    blks.at[ph * 2 + bi // 8].at[
                        :, :, pl.ds((bi % 8) * _BW, _BW)
                    ],
                    sems[ph],
                )

        def process_group(g_scalar, ph):
            for oct_ in range(2):
                pltpu.make_async_copy(
                    tab_hbm.at[:, :, pl.ds(0, 128)],
                    blks.at[ph * 2 + oct_],
                    sems[ph],
                ).wait()
            v16 = idx_v[pl.ds(g_scalar * _GRP, _GRP)]
            k16 = lax.iota(jnp.int32, 16)
            l_vec = (k16 % 8) * _BW + v16 % _BW
            slot = ph * 2 + k16 // 8
            j0 = g_scalar * _GRP
            tcc = j0 // 128
            l2 = j0 % 128
            for d in range(16):
                tr_b = jnp.full((16,), d // 8, jnp.int32)
                s_b = jnp.full((16,), d % 8, jnp.int32)
                vals = plsc.load_gather(blks, [slot, tr_b, s_b, l_vec])
                rows4[d // 8, tcc, d % 8, pl.ds(l2, _GRP)] = vals

        for ph in range(_NPH):
            issue_group(ph, ph)

        def body(gk, carry):
            g = gk * _NPH
            for ph in range(_NPH):
                process_group(g + ph, ph)

                @pl.when(g + ph + _NPH < n_grp)
                def _():
                    issue_group(g + ph + _NPH, ph)

            return carry

        lax.fori_loop(0, n_main // _NPH, body, 0)
        for t in range(n_tail):
            process_group(n_main + t, t)

        for tr in range(2):
            for tcc in range(n_out_blk):
                pltpu.sync_copy(
                    rows4.at[tr, tcc],
                    out_hbm.at[tr].at[:, pl.ds(base + tcc * 128, 128)],
                )

    return k(tab3, idx1)


def kernel(g, h, r, norm, node_attri):
    idx = h.reshape(-1).astype(jnp.int32)
    tab3 = node_attri.T.reshape(2, 8, node_attri.shape[0])
    out3 = _sc_gather_blocks(tab3, idx)
    return out3.reshape(16, idx.shape[0]).T


# speedup vs baseline: 10.3786x; 1.2478x over previous
import functools

import jax
import jax.numpy as jnp
from jax import lax
from jax.experimental import pallas as pl
from jax.experimental.pallas import tpu as pltpu
from jax.experimental.pallas import tpu_sc as plsc

_INFO = plsc.get_sparse_core_info()
_NC = _INFO.num_cores
_NS = _INFO.num_subcores
_NW = _NC * _NS
_GRP = 16
_NPH = 2
_BW = 16


def _sc_gather_blocks(tab3, idx1):
    (b,) = idx1.shape
    b_per_w = b // _NW
    mesh = plsc.VectorSubcoreMesh(core_axis_name="c", subcore_axis_name="s")
    n_out_blk = b_per_w // 128
    n_grp = b_per_w // _GRP
    n_tail = n_grp % _NPH
    n_main = n_grp - n_tail

    @functools.partial(
        pl.kernel,
        mesh=mesh,
        out_type=jax.ShapeDtypeStruct((2, 8, b), jnp.float32),
        scratch_types=(
            [
                pltpu.VMEM((b_per_w,), jnp.int32),
                pltpu.VMEM((_NPH * 2, 2, 8, 128), jnp.float32),
                pltpu.VMEM((2, n_out_blk, 8, 128), jnp.float32),
            ]
            + [pltpu.SemaphoreType.DMA] * _NPH
        ),
        compiler_params=pltpu.CompilerParams(
            disable_bounds_checks=True, needs_layout_passes=False
        ),
    )
    def k(tab_hbm, idx_hbm, out_hbm, idx_v, blks, rows4, *sems):
        wid = lax.axis_index("s") * _NC + lax.axis_index("c")
        base = wid * b_per_w
        pltpu.sync_copy(idx_hbm.at[pl.ds(base, b_per_w)], idx_v)

        def issue_group(g_scalar, ph):
            c16 = (idx_v[pl.ds(g_scalar * _GRP, _GRP)] // _BW) * _BW
            for bi in range(_GRP):
                c0 = pl.multiple_of(c16[bi], _BW)
                pltpu.async_copy(
                    tab_hbm.at[:, :, pl.ds(c0, _BW)],
                    blks.at[ph * 2 + bi // 8].at[
                        :, :, pl.ds((bi % 8) * _BW, _BW)
                    ],
                    sems[ph],
                )

        def process_group(g_scalar, ph):
            for oct_ in range(2):
                pltpu.make_async_copy(
                    tab_hbm.at[:, :, pl.ds(0, 128)],
                    blks.at[ph * 2 + oct_],
                    sems[ph],
                ).wait()
            v16 = idx_v[pl.ds(g_scalar * _GRP, _GRP)]
            k16 = lax.iota(jnp.int32, 16)
            l_vec = (k16 % 8) * _BW + v16 % _BW
            slot = ph * 2 + k16 // 8
            j0 = g_scalar * _GRP
            tcc = j0 // 128
            l2 = j0 % 128
            for d in range(16):
                tr_b = jnp.full((16,), d // 8, jnp.int32)
                s_b = jnp.full((16,), d % 8, jnp.int32)
                vals = plsc.load_gather(blks, [slot, tr_b, s_b, l_vec])
                rows4[d // 8, tcc, d % 8, pl.ds(l2, _GRP)] = vals

        issue_group(0, 0)
        issue_group(1, 1)

        def body(gk, carry):
            g = gk * _NPH
            for ph in range(_NPH):
                process_group(g + ph, ph)
                issue_group(g + ph + _NPH, ph)
            return carry

        lax.fori_loop(0, n_grp // _NPH - 1, body, 0)
        for ph in range(_NPH):
            process_group(n_grp - _NPH + ph, ph)

        out_copies = []
        for tr in range(2):
            for tcc in range(n_out_blk):
                out_copies.append(
                    pltpu.async_copy(
                        rows4.at[tr, tcc],
                        out_hbm.at[tr].at[:, pl.ds(base + tcc * 128, 128)],
                        sems[0],
                    )
                )
        for cp in out_copies:
            cp.wait()

    return k(tab3, idx1)


def kernel(g, h, r, norm, node_attri):
    idx = h.reshape(-1).astype(jnp.int32)
    tab3 = node_attri.T.reshape(2, 8, node_attri.shape[0])
    out3 = _sc_gather_blocks(tab3, idx)
    return out3.reshape(16, idx.shape[0]).T


# speedup vs baseline: 10.3952x; 1.0016x over previous
import functools

import jax
import jax.numpy as jnp
from jax import lax
from jax.experimental import pallas as pl
from jax.experimental.pallas import tpu as pltpu
from jax.experimental.pallas import tpu_sc as plsc

_INFO = plsc.get_sparse_core_info()
_NC = _INFO.num_cores
_NS = _INFO.num_subcores
_NW = _NC * _NS
_GRP = 16
_NPH = 2
_BW = 16


def _sc_gather_blocks(tab3, idx1):
    (b,) = idx1.shape
    b_per_w = b // _NW
    mesh = plsc.VectorSubcoreMesh(core_axis_name="c", subcore_axis_name="s")
    n_out_blk = b_per_w // 128
    n_grp = b_per_w // _GRP
    assert n_grp % _NPH == 0 and b % (_NW * _GRP) == 0

    @functools.partial(
        pl.kernel,
        mesh=mesh,
        out_type=jax.ShapeDtypeStruct((2, 8, b), jnp.float32),
        scratch_types=(
            [
                pltpu.VMEM((b_per_w,), jnp.int32),
                pltpu.VMEM((_NPH * 2, 2, 8, 128), jnp.float32),
                pltpu.VMEM((2, n_out_blk, 8, 128), jnp.float32),
            ]
            + [pltpu.SemaphoreType.DMA] * _NPH
        ),
        compiler_params=pltpu.CompilerParams(
            disable_bounds_checks=True, needs_layout_passes=False
        ),
    )
    def k(tab_hbm, idx_hbm, out_hbm, idx_v, blks, rows4, *sems):
        wid = lax.axis_index("s") * _NC + lax.axis_index("c")
        base = wid * b_per_w
        pltpu.sync_copy(idx_hbm.at[pl.ds(base, b_per_w)], idx_v)

        def issue_group(g_scalar, ph):
            c16 = (idx_v[pl.ds(g_scalar * _GRP, _GRP)] // _BW) * _BW
            for bi in range(_GRP):
                c0 = pl.multiple_of(c16[bi], _BW)
                pltpu.async_copy(
                    tab_hbm.at[:, :, pl.ds(c0, _BW)],
                    blks.at[ph * 2 + bi // 8].at[
                        :, :, pl.ds((bi % 8) * _BW, _BW)
                    ],
                    sems[ph],
                )

        def process_group(g_scalar, ph):
            for oct_ in range(2):
                pltpu.make_async_copy(
                    tab_hbm.at[:, :, pl.ds(0, 128)],
                    blks.at[ph * 2 + oct_],
                    sems[ph],
                ).wait()
            v16 = idx_v[pl.ds(g_scalar * _GRP, _GRP)]
            k16 = lax.iota(jnp.int32, 16)
            l_vec = (k16 % 8) * _BW + v16 % _BW
            slot = ph * 2 + k16 // 8
            j0 = g_scalar * _GRP
            tcc = j0 // 128
            l2 = j0 % 128
            for d in range(16):
                tr_b = jnp.full((16,), d // 8, jnp.int32)
                s_b = jnp.full((16,), d % 8, jnp.int32)
                vals = plsc.load_gather(blks, [slot, tr_b, s_b, l_vec])
                rows4[d // 8, tcc, d % 8, pl.ds(l2, _GRP)] = vals

        issue_group(0, 0)
        issue_group(1, 1)

        def body(gk, carry):
            g = gk * _NPH
            for ph in range(_NPH):
                process_group(g + ph, ph)
                issue_group(g + ph + _NPH, ph)
            return carry

        lax.fori_loop(0, n_grp // _NPH - 1, body, 0)
        for ph in range(_NPH):
            process_group(n_grp - _NPH + ph, ph)

        out_copies = []
        for tr in range(2):
            for tcc in range(n_out_blk):
                out_copies.append(
                    pltpu.async_copy(
                        rows4.at[tr, tcc],
                        out_hbm.at[tr].at[:, pl.ds(base + tcc * 128, 128)],
                        sems[0],
                    )
                )
        for cp in out_copies:
            cp.wait()

    return k(tab3, idx1)


def kernel(g, h, r, norm, node_attri):
    idx = h.reshape(-1).astype(jnp.int32)
    tab3 = node_attri.T.reshape(2, 8, node_attri.shape[0])
    out3 = _sc_gather_blocks(tab3, idx)
    return out3.reshape(16, idx.shape[0]).T
